# baseline (speedup 1.0000x reference)
import functools

import jax
import jax.numpy as jnp
from jax import lax
from jax.experimental import pallas as pl
from jax.experimental.pallas import tpu as pltpu
from jax.experimental.pallas import tpu_sc as plsc

_NC = 2
_NS = 16
_NW = _NC * _NS
_L = 16
_CW = 128
_TCH = 80
_ZR = 80


def _sc_mesh():
    return plsc.VectorSubcoreMesh(core_axis_name="c", subcore_axis_name="s")


def _pad_rows(row, n):
    e = row.shape[0]
    ept = e // _NW
    padw = _TCH * _CW - ept
    row2 = row.reshape(_NW, ept)
    dummy = n + (jnp.arange(padw, dtype=row.dtype) % 40)
    rowp = jnp.concatenate(
        [row2, jnp.broadcast_to(dummy, (_NW, padw))], axis=1)
    return rowp.reshape(_NW * _TCH, _CW)


def _deg_partials(rowp, n):
    na = n + _ZR
    nzc = na // _ZR
    nwb = n // _ZR

    @functools.partial(
        pl.kernel,
        out_type=jax.ShapeDtypeStruct((_NC, n, _L), jnp.float32),
        mesh=_sc_mesh(),
        compiler_params=pltpu.CompilerParams(use_tc_tiling_on_sc=False),
        scratch_types=[
            pltpu.VMEM((_TCH, _CW), jnp.int32),
            pltpu.VMEM((_CW, _L), jnp.float32),
            pltpu.VMEM((_ZR, _L), jnp.float32),
            pltpu.VMEM_SHARED((na, _L), jnp.float32),
            pltpu.SemaphoreType.DMA,
        ],
    )
    def deg_kernel(row_hbm, out_hbm, rowA, ones_v, zeros_v, acc_sh, sem):
        cid = lax.axis_index("c")
        sid = lax.axis_index("s")
        wid = cid * _NS + sid

        pltpu.sync_copy(row_hbm.at[pl.ds(wid * _TCH, _TCH)], rowA)

        @pl.loop(0, _CW)
        def _(i):
            ones_v[i, :] = jnp.ones((_L,), jnp.float32)

        @pl.loop(0, _ZR)
        def _(i):
            zeros_v[i, :] = jnp.zeros((_L,), jnp.float32)

        @pl.loop(sid, nzc, step=_NS)
        def _(j):
            pltpu.sync_copy(zeros_v, acc_sh.at[pl.ds(j * _ZR, _ZR)])

        plsc.subcore_barrier()

        @pl.loop(0, _TCH // 8)
        def _(k):
            @pl.loop(0, 8)
            def _(j):
                pltpu.async_copy(ones_v, acc_sh.at[rowA.at[k * 8 + j]], sem,
                                 add=True)

            @pl.loop(0, 8)
            def _(j):
                pltpu.make_async_copy(ones_v, acc_sh.at[rowA.at[k * 8 + j]],
                                      sem).wait()

        plsc.subcore_barrier()

        @pl.loop(sid, nwb, step=_NS)
        def _(j):
            pltpu.sync_copy(acc_sh.at[pl.ds(j * _ZR, _ZR)],
                            out_hbm.at[cid, pl.ds(j * _ZR, _ZR)])

    return deg_kernel(rowp)


def _edge_partials(hs, colp, rowp):
    n, d = hs.shape
    nch = rowp.shape[1]
    dk = rowp.shape[2]
    nzc = n // dk

    @functools.partial(
        pl.kernel,
        out_type=jax.ShapeDtypeStruct((_NC, n, d), jnp.float32),
        mesh=_sc_mesh(),
        scratch_types=[
            pltpu.VMEM((nch * dk,), jnp.int32),
            pltpu.VMEM((nch, dk), jnp.int32),
            pltpu.VMEM((2, dk, d), jnp.float32),
            pltpu.VMEM_SHARED((n, d), jnp.float32),
            pltpu.SemaphoreType.DMA,
            pltpu.SemaphoreType.DMA,
        ],
    )
    def pump_kernel(hs_hbm, col_hbm, row_hbm, out_hbm,
                    colA, rowA, bufs, acc_sh, sem0, sem1):
        cid = lax.axis_index("c")
        sid = lax.axis_index("s")
        wid = cid * _NS + sid

        pltpu.sync_copy(col_hbm.at[wid], colA)
        pltpu.sync_copy(row_hbm.at[wid], rowA)

        @pl.loop(0, dk)
        def _(i):
            @pl.loop(0, d // _L)
            def _(j):
                bufs[0, i, pl.ds(j * _L, _L)] = jnp.zeros((_L,), jnp.float32)

        @pl.loop(sid, nzc, step=_NS)
        def _(j):
            pltpu.sync_copy(bufs.at[0], acc_sh.at[pl.ds(j * dk, dk)])

        plsc.subcore_barrier()

        def gstart(c, b, sem):
            pltpu.async_copy(hs_hbm.at[colA.at[pl.ds(c * dk, dk)]],
                             bufs.at[b], sem)

        def gwait(c, b, sem):
            pltpu.make_async_copy(hs_hbm.at[colA.at[pl.ds(c * dk, dk)]],
                                  bufs.at[b], sem).wait()

        def scat(c, b):
            pltpu.sync_copy(bufs.at[b], acc_sh.at[rowA.at[c]], add=True)

        gstart(0, 0, sem0)
        gstart(1, 1, sem1)

        @pl.loop(0, (nch - 1) // 2)
        def _(p):
            c = 2 * p
            gwait(c, 0, sem0)
            scat(c, 0)
            gstart(c + 2, 0, sem0)
            gwait(c + 1, 1, sem1)
            scat(c + 1, 1)

            @pl.when(c + 3 < nch)
            def _():
                gstart(c + 3, 1, sem1)

        gwait(nch - 1, 0, sem0)
        scat(nch - 1, 0)

        plsc.subcore_barrier()

        @pl.loop(sid, nzc, step=_NS)
        def _(j):
            pltpu.sync_copy(acc_sh.at[pl.ds(j * dk, dk)],
                            out_hbm.at[cid, pl.ds(j * dk, dk)])

    return pump_kernel(hs, colp, rowp)


def _matmul_scale(x, weight, degp):
    n, din = x.shape
    dout = weight.shape[1]
    blk = 1000

    def body(x_ref, w_ref, d_ref, o_ref):
        h = jnp.dot(x_ref[...], w_ref[...],
                    preferred_element_type=jnp.float32)
        deg = d_ref[0, :, 0:1] + d_ref[1, :, 0:1] + 1.0
        o_ref[...] = h * lax.rsqrt(deg)

    return pl.pallas_call(
        body,
        grid=(n // blk,),
        in_specs=[
            pl.BlockSpec((blk, din), lambda i: (i, 0)),
            pl.BlockSpec((din, dout), lambda i: (0, 0)),
            pl.BlockSpec((_NC, blk, _L), lambda i: (0, i, 0)),
        ],
        out_specs=pl.BlockSpec((blk, dout), lambda i: (i, 0)),
        out_shape=jax.ShapeDtypeStruct((n, dout), jnp.float32),
    )(x, weight, degp)


def _finish(hs, accp, degp, bias):
    n, d = hs.shape
    blk = 1000

    def body(hs_ref, a_ref, d_ref, b_ref, o_ref):
        deg = d_ref[0, :, 0:1] + d_ref[1, :, 0:1] + 1.0
        nrm = lax.rsqrt(deg)
        o_ref[...] = nrm * (a_ref[0] + a_ref[1] + hs_ref[...]) + b_ref[...]

    return pl.pallas_call(
        body,
        grid=(n // blk,),
        in_specs=[
            pl.BlockSpec((blk, d), lambda i: (i, 0)),
            pl.BlockSpec((_NC, blk, d), lambda i: (0, i, 0)),
            pl.BlockSpec((_NC, blk, _L), lambda i: (0, i, 0)),
            pl.BlockSpec((1, d), lambda i: (0, 0)),
        ],
        out_specs=pl.BlockSpec((blk, d), lambda i: (i, 0)),
        out_shape=jax.ShapeDtypeStruct((n, d), jnp.float32),
    )(hs, accp, degp, bias.reshape(1, d))


def kernel(x, edge_index, weight, bias):
    row = edge_index[0]
    col = edge_index[1]
    n = x.shape[0]
    e = row.shape[0]
    dk = 80
    nch = e // (_NW * dk)
    rowpd = _pad_rows(row, n)
    colp = col.reshape(_NW, nch * dk)
    rowp = row.reshape(_NW, nch, dk)
    degp = _deg_partials(rowpd, n)
    hs = _matmul_scale(x, weight, degp)
    accp = _edge_partials(hs, colp, rowp)
    return _finish(hs, accp, degp, bias)


# speedup vs baseline: 1.0581x; 1.0581x over previous
import functools

import jax
import jax.numpy as jnp
from jax import lax
from jax.experimental import pallas as pl
from jax.experimental.pallas import tpu as pltpu
from jax.experimental.pallas import tpu_sc as plsc

_NC = 2
_NS = 16
_NW = _NC * _NS
_L = 16
_CW = 128
_TCH = 80
_ZR = 80


def _sc_mesh():
    return plsc.VectorSubcoreMesh(core_axis_name="c", subcore_axis_name="s")


def _deg_partials(edge_index, n):
    e = edge_index.shape[1]
    ept = e // _NW
    dk = 80
    nch = ept // dk
    nzc = n // _ZR

    @functools.partial(
        pl.kernel,
        out_type=jax.ShapeDtypeStruct((_NC, n, _L), jnp.float32),
        mesh=_sc_mesh(),
        compiler_params=pltpu.CompilerParams(use_tc_tiling_on_sc=False),
        scratch_types=[
            pltpu.VMEM((ept,), jnp.int32),
            pltpu.VMEM((dk, _L), jnp.float32),
            pltpu.VMEM((_ZR, _L), jnp.float32),
            pltpu.VMEM_SHARED((n, _L), jnp.float32),
            pltpu.SemaphoreType.DMA,
        ],
    )
    def deg_kernel(ei_hbm, out_hbm, rowA, ones_v, zeros_v, acc_sh, sem):
        cid = lax.axis_index("c")
        sid = lax.axis_index("s")
        wid = cid * _NS + sid

        pltpu.sync_copy(ei_hbm.at[0, pl.ds(wid * ept, ept)], rowA)

        @pl.loop(0, dk)
        def _(i):
            ones_v[i, :] = jnp.ones((_L,), jnp.float32)

        @pl.loop(0, _ZR)
        def _(i):
            zeros_v[i, :] = jnp.zeros((_L,), jnp.float32)

        @pl.loop(sid, nzc, step=_NS)
        def _(j):
            pltpu.sync_copy(zeros_v, acc_sh.at[pl.ds(j * _ZR, _ZR)])

        plsc.subcore_barrier()

        @pl.loop(0, nch // 5)
        def _(k):
            @pl.loop(0, 5)
            def _(j):
                c = k * 5 + j
                pltpu.async_copy(ones_v, acc_sh.at[rowA.at[pl.ds(c * dk, dk)]],
                                 sem, add=True)

            @pl.loop(0, 5)
            def _(j):
                c = k * 5 + j
                pltpu.make_async_copy(ones_v,
                                      acc_sh.at[rowA.at[pl.ds(c * dk, dk)]],
                                      sem).wait()

        plsc.subcore_barrier()

        @pl.loop(sid, nzc, step=_NS)
        def _(j):
            pltpu.sync_copy(acc_sh.at[pl.ds(j * _ZR, _ZR)],
                            out_hbm.at[cid, pl.ds(j * _ZR, _ZR)])

    return deg_kernel(edge_index)


def _edge_partials(hs, edge_index):
    n, d = hs.shape
    e = edge_index.shape[1]
    ept = e // _NW
    dk = 80
    nch = ept // dk
    nzc = n // dk

    @functools.partial(
        pl.kernel,
        out_type=jax.ShapeDtypeStruct((_NC, n, d), jnp.float32),
        mesh=_sc_mesh(),
        compiler_params=pltpu.CompilerParams(use_tc_tiling_on_sc=False),
        scratch_types=[
            pltpu.VMEM((ept,), jnp.int32),
            pltpu.VMEM((ept,), jnp.int32),
            pltpu.VMEM((2, dk, d), jnp.float32),
            pltpu.VMEM_SHARED((n, d), jnp.float32),
            pltpu.SemaphoreType.DMA,
            pltpu.SemaphoreType.DMA,
        ],
    )
    def pump_kernel(hs_hbm, ei_hbm, out_hbm,
                    colA, rowA, bufs, acc_sh, sem0, sem1):
        cid = lax.axis_index("c")
        sid = lax.axis_index("s")
        wid = cid * _NS + sid

        pltpu.sync_copy(ei_hbm.at[1, pl.ds(wid * ept, ept)], colA)
        pltpu.sync_copy(ei_hbm.at[0, pl.ds(wid * ept, ept)], rowA)

        @pl.loop(0, dk)
        def _(i):
            @pl.loop(0, d // _L)
            def _(j):
                bufs[0, i, pl.ds(j * _L, _L)] = jnp.zeros((_L,), jnp.float32)

        @pl.loop(sid, nzc, step=_NS)
        def _(j):
            pltpu.sync_copy(bufs.at[0], acc_sh.at[pl.ds(j * dk, dk)])

        plsc.subcore_barrier()

        def gstart(c, b, sem):
            pltpu.async_copy(hs_hbm.at[colA.at[pl.ds(c * dk, dk)]],
                             bufs.at[b], sem)

        def gwait(c, b, sem):
            pltpu.make_async_copy(hs_hbm.at[colA.at[pl.ds(c * dk, dk)]],
                                  bufs.at[b], sem).wait()

        def scat(c, b):
            pltpu.sync_copy(bufs.at[b], acc_sh.at[rowA.at[pl.ds(c * dk, dk)]],
                            add=True)

        gstart(0, 0, sem0)
        gstart(1, 1, sem1)

        @pl.loop(0, (nch - 1) // 2)
        def _(p):
            c = 2 * p
            gwait(c, 0, sem0)
            scat(c, 0)
            gstart(c + 2, 0, sem0)
            gwait(c + 1, 1, sem1)
            scat(c + 1, 1)

            @pl.when(c + 3 < nch)
            def _():
                gstart(c + 3, 1, sem1)

        gwait(nch - 1, 0, sem0)
        scat(nch - 1, 0)

        plsc.subcore_barrier()

        @pl.loop(sid, nzc, step=_NS)
        def _(j):
            pltpu.sync_copy(acc_sh.at[pl.ds(j * dk, dk)],
                            out_hbm.at[cid, pl.ds(j * dk, dk)])

    return pump_kernel(hs, edge_index)


def _matmul_scale(x, weight, degp):
    n, din = x.shape
    dout = weight.shape[1]
    blk = 1000

    def body(x_ref, w_ref, d_ref, o_ref):
        h = jnp.dot(x_ref[...], w_ref[...],
                    preferred_element_type=jnp.float32)
        deg = d_ref[0, :, 0:1] + d_ref[1, :, 0:1] + 1.0
        o_ref[...] = h * lax.rsqrt(deg)

    return pl.pallas_call(
        body,
        grid=(n // blk,),
        in_specs=[
            pl.BlockSpec((blk, din), lambda i: (i, 0)),
            pl.BlockSpec((din, dout), lambda i: (0, 0)),
            pl.BlockSpec((_NC, blk, _L), lambda i: (0, i, 0)),
        ],
        out_specs=pl.BlockSpec((blk, dout), lambda i: (i, 0)),
        out_shape=jax.ShapeDtypeStruct((n, dout), jnp.float32),
    )(x, weight, degp)


def _finish(hs, accp, degp, bias):
    n, d = hs.shape
    blk = 1000

    def body(hs_ref, a_ref, d_ref, b_ref, o_ref):
        deg = d_ref[0, :, 0:1] + d_ref[1, :, 0:1] + 1.0
        nrm = lax.rsqrt(deg)
        o_ref[...] = nrm * (a_ref[0] + a_ref[1] + hs_ref[...]) + b_ref[...]

    return pl.pallas_call(
        body,
        grid=(n // blk,),
        in_specs=[
            pl.BlockSpec((blk, d), lambda i: (i, 0)),
            pl.BlockSpec((_NC, blk, d), lambda i: (0, i, 0)),
            pl.BlockSpec((_NC, blk, _L), lambda i: (0, i, 0)),
            pl.BlockSpec((1, d), lambda i: (0, 0)),
        ],
        out_specs=pl.BlockSpec((blk, d), lambda i: (i, 0)),
        out_shape=jax.ShapeDtypeStruct((n, d), jnp.float32),
    )(hs, accp, degp, bias.reshape(1, d))


def kernel(x, edge_index, weight, bias):
    n = x.shape[0]
    degp = _deg_partials(edge_index, n)
    hs = _matmul_scale(x, weight, degp)
    accp = _edge_partials(hs, edge_index)
    return _finish(hs, accp, degp, bias)


# speedup vs baseline: 1.2007x; 1.1348x over previous
import functools

import jax
import jax.numpy as jnp
from jax import lax
from jax.experimental import pallas as pl
from jax.experimental.pallas import tpu as pltpu
from jax.experimental.pallas import tpu_sc as plsc

_NC = 2
_NS = 16
_NW = _NC * _NS
_L = 16
_CW = 128
_TCH = 80
_ZR = 80


def _sc_mesh():
    return plsc.VectorSubcoreMesh(core_axis_name="c", subcore_axis_name="s")


def _deg_partials(edge_index, n):
    e = edge_index.shape[1]
    ept = e // _NW
    dk = 80
    nch = ept // dk
    nzc = n // _ZR

    @functools.partial(
        pl.kernel,
        out_type=jax.ShapeDtypeStruct((_NC, n, _L), jnp.float32),
        mesh=_sc_mesh(),
        compiler_params=pltpu.CompilerParams(use_tc_tiling_on_sc=False),
        scratch_types=[
            pltpu.VMEM((ept,), jnp.int32),
            pltpu.VMEM((dk, _L), jnp.float32),
            pltpu.VMEM((_ZR, _L), jnp.float32),
            pltpu.VMEM_SHARED((n, _L), jnp.float32),
            pltpu.SemaphoreType.DMA,
        ],
    )
    def deg_kernel(ei_hbm, out_hbm, rowA, ones_v, zeros_v, acc_sh, sem):
        cid = lax.axis_index("c")
        sid = lax.axis_index("s")
        wid = cid * _NS + sid

        pltpu.sync_copy(ei_hbm.at[0, pl.ds(wid * ept, ept)], rowA)

        @pl.loop(0, dk)
        def _(i):
            ones_v[i, :] = jnp.ones((_L,), jnp.float32)

        @pl.loop(0, _ZR)
        def _(i):
            zeros_v[i, :] = jnp.zeros((_L,), jnp.float32)

        @pl.loop(sid, nzc, step=_NS)
        def _(j):
            pltpu.sync_copy(zeros_v, acc_sh.at[pl.ds(j * _ZR, _ZR)])

        plsc.subcore_barrier()

        @pl.loop(0, nch // 5)
        def _(k):
            @pl.loop(0, 5)
            def _(j):
                c = k * 5 + j
                pltpu.async_copy(ones_v, acc_sh.at[rowA.at[pl.ds(c * dk, dk)]],
                                 sem, add=True)

            @pl.loop(0, 5)
            def _(j):
                c = k * 5 + j
                pltpu.make_async_copy(ones_v,
                                      acc_sh.at[rowA.at[pl.ds(c * dk, dk)]],
                                      sem).wait()

        plsc.subcore_barrier()

        @pl.loop(sid, nzc, step=_NS)
        def _(j):
            pltpu.sync_copy(acc_sh.at[pl.ds(j * _ZR, _ZR)],
                            out_hbm.at[cid, pl.ds(j * _ZR, _ZR)])

    return deg_kernel(edge_index)


def _edge_partials(hs, edge_index):
    n, d = hs.shape
    e = edge_index.shape[1]
    ept = e // _NW
    dk = 80
    nch = ept // dk
    nzc = n // dk

    @functools.partial(
        pl.kernel,
        out_type=jax.ShapeDtypeStruct((_NC, n, d), jnp.float32),
        mesh=_sc_mesh(),
        compiler_params=pltpu.CompilerParams(use_tc_tiling_on_sc=False),
        scratch_types=[
            pltpu.VMEM((ept,), jnp.int32),
            pltpu.VMEM((ept,), jnp.int32),
            pltpu.VMEM((3, dk, d), jnp.float32),
            pltpu.VMEM_SHARED((n, d), jnp.float32),
            pltpu.SemaphoreType.DMA,
            pltpu.SemaphoreType.DMA,
            pltpu.SemaphoreType.DMA,
        ],
    )
    def pump_kernel(hs_hbm, ei_hbm, out_hbm,
                    colA, rowA, bufs, acc_sh, sem0, sem1, sem2):
        cid = lax.axis_index("c")
        sid = lax.axis_index("s")
        wid = cid * _NS + sid

        pltpu.sync_copy(ei_hbm.at[1, pl.ds(wid * ept, ept)], colA)
        pltpu.sync_copy(ei_hbm.at[0, pl.ds(wid * ept, ept)], rowA)

        @pl.loop(0, dk)
        def _(i):
            @pl.loop(0, d // _L)
            def _(j):
                bufs[0, i, pl.ds(j * _L, _L)] = jnp.zeros((_L,), jnp.float32)

        @pl.loop(sid, nzc, step=_NS)
        def _(j):
            pltpu.sync_copy(bufs.at[0], acc_sh.at[pl.ds(j * dk, dk)])

        plsc.subcore_barrier()

        def gstart(c, b, sem):
            pltpu.async_copy(hs_hbm.at[colA.at[pl.ds(c * dk, dk)]],
                             bufs.at[b], sem)

        def gwait(c, b, sem):
            pltpu.make_async_copy(hs_hbm.at[colA.at[pl.ds(c * dk, dk)]],
                                  bufs.at[b], sem).wait()

        def scat(c, b):
            pltpu.sync_copy(bufs.at[b], acc_sh.at[rowA.at[pl.ds(c * dk, dk)]],
                            add=True)

        sems = (sem0, sem1, sem2)
        nb = 3
        body = (nch // nb - 1) * nb

        for b in range(nb):
            gstart(b, b, sems[b])

        @pl.loop(0, body // nb)
        def _(p):
            for k in range(nb):
                c = nb * p + k
                gwait(c, k, sems[k])
                scat(c, k)
                gstart(c + nb, k, sems[k])

        for c0 in range(body, nch):
            b = c0 % nb
            gwait(c0, b, sems[b])
            scat(c0, b)
            if c0 + nb < nch:
                gstart(c0 + nb, b, sems[b])

        plsc.subcore_barrier()

        @pl.loop(sid, nzc, step=_NS)
        def _(j):
            pltpu.sync_copy(acc_sh.at[pl.ds(j * dk, dk)],
                            out_hbm.at[cid, pl.ds(j * dk, dk)])

    return pump_kernel(hs, edge_index)


def _matmul_scale(x, weight, degp):
    n, din = x.shape
    dout = weight.shape[1]
    blk = 1000

    def body(x_ref, w_ref, d_ref, o_ref):
        h = jnp.dot(x_ref[...], w_ref[...],
                    preferred_element_type=jnp.float32)
        deg = d_ref[0, :, 0:1] + d_ref[1, :, 0:1] + 1.0
        o_ref[...] = h * lax.rsqrt(deg)

    return pl.pallas_call(
        body,
        grid=(n // blk,),
        in_specs=[
            pl.BlockSpec((blk, din), lambda i: (i, 0)),
            pl.BlockSpec((din, dout), lambda i: (0, 0)),
            pl.BlockSpec((_NC, blk, _L), lambda i: (0, i, 0)),
        ],
        out_specs=pl.BlockSpec((blk, dout), lambda i: (i, 0)),
        out_shape=jax.ShapeDtypeStruct((n, dout), jnp.float32),
    )(x, weight, degp)


def _finish(hs, accp, degp, bias):
    n, d = hs.shape
    blk = 1000

    def body(hs_ref, a_ref, d_ref, b_ref, o_ref):
        deg = d_ref[0, :, 0:1] + d_ref[1, :, 0:1] + 1.0
        nrm = lax.rsqrt(deg)
        o_ref[...] = nrm * (a_ref[0] + a_ref[1] + hs_ref[...]) + b_ref[...]

    return pl.pallas_call(
        body,
        grid=(n // blk,),
        in_specs=[
            pl.BlockSpec((blk, d), lambda i: (i, 0)),
            pl.BlockSpec((_NC, blk, d), lambda i: (0, i, 0)),
            pl.BlockSpec((_NC, blk, _L), lambda i: (0, i, 0)),
            pl.BlockSpec((1, d), lambda i: (0, 0)),
        ],
        out_specs=pl.BlockSpec((blk, d), lambda i: (i, 0)),
        out_shape=jax.ShapeDtypeStruct((n, d), jnp.float32),
    )(hs, accp, degp, bias.reshape(1, d))


def kernel(x, edge_index, weight, bias):
    n = x.shape[0]
    degp = _deg_partials(edge_index, n)
    hs = _matmul_scale(x, weight, degp)
    accp = _edge_partials(hs, edge_index)
    return _finish(hs, accp, degp, bias)
